# baseline (speedup 1.0000x reference)
import functools

import jax
import jax.numpy as jnp
from jax import lax
from jax.experimental import pallas as pl
from jax.experimental.pallas import tpu as pltpu
from jax.experimental.pallas import tpu_sc as plsc

_BATCH = 16384
_NF = 26
_VOCAB = 100000
_EMB = 16

_NC = 2
_NS = 16
_NW = _NC * _NS
_ROWS = _BATCH * _NF
_RPW = _ROWS // _NW
_CHUNK = 1664
_NCHUNK = _RPW // _CHUNK
_VPC = _CHUNK // 16


@functools.partial(
    pl.kernel,
    out_type=jax.ShapeDtypeStruct((_ROWS, _EMB), jnp.float32),
    mesh=plsc.VectorSubcoreMesh(core_axis_name="c", subcore_axis_name="s"),
    compiler_params=pltpu.CompilerParams(use_tc_tiling_on_sc=False),
    scratch_types=[
        pltpu.VMEM((_CHUNK,), jnp.int32),
        pltpu.VMEM((_CHUNK,), jnp.int32),
        pltpu.VMEM((_CHUNK,), jnp.int32),
        pltpu.VMEM((_CHUNK, _EMB), jnp.float32),
        pltpu.SemaphoreType.DMA,
    ],
)
def _sc_gather(xf, off, tab, out, xbuf, offbuf, idxbuf, rowbuf, sem):
    wid = lax.axis_index("s") * _NC + lax.axis_index("c")
    base_w = wid * _RPW
    pltpu.sync_copy(off, offbuf)

    def chunk(c, carry):
        base = base_w + c * _CHUNK
        pltpu.sync_copy(xf.at[pl.ds(base, _CHUNK)], xbuf)
        for v in range(_VPC):
            sl = pl.ds(v * 16, 16)
            idxbuf[sl] = xbuf[sl] + offbuf[sl]
        pltpu.async_copy(tab.at[idxbuf], rowbuf, sem).wait()
        pltpu.sync_copy(rowbuf, out.at[pl.ds(base, _CHUNK)])
        return carry

    lax.fori_loop(0, _NCHUNK, chunk, 0)


def kernel(x, tables):
    xf = x.astype(jnp.int32).reshape(_ROWS)
    tab = tables.reshape(_NF * _VOCAB, _EMB)
    off = jnp.tile(jnp.arange(_NF, dtype=jnp.int32) * _VOCAB, _CHUNK // _NF)
    out = _sc_gather(xf, off, tab)
    return out.reshape(_BATCH, _NF * _EMB)


# speedup vs baseline: 5.3510x; 5.3510x over previous
import functools

import jax
import jax.numpy as jnp
from jax import lax
from jax.experimental import pallas as pl
from jax.experimental.pallas import tpu as pltpu
from jax.experimental.pallas import tpu_sc as plsc

_BATCH = 16384
_NF = 26
_VOCAB = 100000
_EMB = 16

_NC = 2
_NS = 16
_NW = _NC * _NS
_NPLANE = _NF * _EMB
_TPW = _NPLANE // _NW
_HB = _BATCH // 2


@functools.partial(
    pl.kernel,
    out_type=jax.ShapeDtypeStruct((_NPLANE, _BATCH), jnp.float32),
    mesh=plsc.VectorSubcoreMesh(core_axis_name="c", subcore_axis_name="s"),
    compiler_params=pltpu.CompilerParams(
        use_tc_tiling_on_sc=True, needs_layout_passes=False),
    scratch_types=[
        pltpu.VMEM((_VOCAB,), jnp.float32),
        pltpu.VMEM((_BATCH,), jnp.int32),
        pltpu.VMEM((_HB,), jnp.float32),
    ],
)
def _sc_plane_gather(tabt, xt, out, planebuf, idxbuf, outbuf):
    wid = lax.axis_index("s") * _NC + lax.axis_index("c")

    def task(t, carry):
        pid = wid * _TPW + t
        f = pid // _EMB
        e = pid % _EMB
        pltpu.sync_copy(xt.at[f], idxbuf)
        pltpu.sync_copy(tabt.at[f, e], planebuf)
        for h in range(2):
            def gat(j, c2):
                iv = idxbuf[pl.ds(h * _HB + j * 16, 16)]
                outbuf[pl.ds(j * 16, 16)] = plsc.load_gather(planebuf, [iv])
                return c2
            lax.fori_loop(0, _HB // 16, gat, 0)
            pltpu.sync_copy(outbuf, out.at[pid, pl.ds(h * _HB, _HB)])
        return carry

    lax.fori_loop(0, _TPW, task, 0)


def kernel(x, tables):
    tabt = tables.transpose(0, 2, 1)
    xt = x.astype(jnp.int32).T
    out_t = _sc_plane_gather(tabt, xt)
    return out_t.T


# speedup vs baseline: 8.8279x; 1.6497x over previous
import functools

import jax
import jax.numpy as jnp
from jax import lax
from jax.experimental import pallas as pl
from jax.experimental.pallas import tpu as pltpu
from jax.experimental.pallas import tpu_sc as plsc

_BATCH = 16384
_NF = 26
_VOCAB = 100000
_EMB = 16

_NC = 2
_NS = 16
_NW = _NC * _NS
_NPLANE = _NF * _EMB
_TPW = _NPLANE // _NW
_HB = _BATCH // 2


@functools.partial(
    pl.kernel,
    out_type=jax.ShapeDtypeStruct((_NPLANE, _BATCH), jnp.float32),
    mesh=plsc.VectorSubcoreMesh(core_axis_name="c", subcore_axis_name="s"),
    compiler_params=pltpu.CompilerParams(
        use_tc_tiling_on_sc=True, needs_layout_passes=False),
    scratch_types=[
        pltpu.VMEM((_VOCAB,), jnp.float32),
        pltpu.VMEM((_BATCH,), jnp.int32),
        pltpu.VMEM((_HB,), jnp.float32),
    ],
)
def _sc_plane_gather(tabt, xt, out, planebuf, idxbuf, outbuf):
    wid = lax.axis_index("s") * _NC + lax.axis_index("c")

    def task(t, carry):
        pid = wid * _TPW + t
        f = pid // _EMB
        e = pid % _EMB
        pltpu.sync_copy(xt.at[f], idxbuf)
        pltpu.sync_copy(tabt.at[f, e], planebuf)
        for h in range(2):
            @plsc.parallel_loop(0, _HB, 16, unroll=8)
            def gat(i):
                iv = idxbuf[pl.ds(h * _HB + i, 16)]
                outbuf[pl.ds(i, 16)] = plsc.load_gather(planebuf, [iv])
            pltpu.sync_copy(outbuf, out.at[pid, pl.ds(h * _HB, _HB)])
        return carry

    lax.fori_loop(0, _TPW, task, 0)


def kernel(x, tables):
    tabt = tables.transpose(0, 2, 1)
    xt = x.astype(jnp.int32).T
    out_t = _sc_plane_gather(tabt, xt)
    return out_t.T


# speedup vs baseline: 10.1872x; 1.1540x over previous
import functools

import jax
import jax.numpy as jnp
from jax import lax
from jax.experimental import pallas as pl
from jax.experimental.pallas import tpu as pltpu
from jax.experimental.pallas import tpu_sc as plsc

_BATCH = 16384
_NF = 26
_VOCAB = 100000
_EMB = 16

_NC = 2
_NS = 16
_NW = _NC * _NS
_NPLANE = _NF * _EMB
_TPW = _NPLANE // _NW
_QB = _BATCH // 4


@functools.partial(
    pl.kernel,
    out_type=jax.ShapeDtypeStruct((_NPLANE, _BATCH), jnp.float32),
    mesh=plsc.VectorSubcoreMesh(core_axis_name="c", subcore_axis_name="s"),
    compiler_params=pltpu.CompilerParams(
        use_tc_tiling_on_sc=True, needs_layout_passes=False),
    scratch_types=[
        pltpu.VMEM((_VOCAB,), jnp.float32),
        pltpu.VMEM((_BATCH,), jnp.int32),
        pltpu.VMEM((_QB,), jnp.float32),
        pltpu.VMEM((_QB,), jnp.float32),
        pltpu.SemaphoreType.DMA,
        pltpu.SemaphoreType.DMA,
        pltpu.SemaphoreType.DMA,
    ],
)
def _sc_plane_gather(tabt, xt, out, planebuf, idxbuf, qbuf0, qbuf1, semp, sem0, sem1):
    wid = lax.axis_index("s") * _NC + lax.axis_index("c")
    qbufs, sems = (qbuf0, qbuf1), (sem0, sem1)
    inflight = [None, None]
    qcount = 0

    for t in range(_TPW):
        pid = wid * _TPW + t
        f = pid // _EMB
        e = pid % _EMB
        plane_cp = pltpu.async_copy(tabt.at[f, e], planebuf, semp)
        if t == 0:
            pltpu.sync_copy(xt.at[f], idxbuf)
        else:
            f_prev = (pid - 1) // _EMB
            @pl.when(f != f_prev)
            def _load_idx():
                pltpu.sync_copy(xt.at[f], idxbuf)
        plane_cp.wait()
        for q in range(4):
            qb = qcount % 2
            if inflight[qb] is not None:
                inflight[qb].wait()
            outbuf = qbufs[qb]

            @plsc.parallel_loop(0, _QB, 16, unroll=8)
            def gat(i):
                iv = idxbuf[pl.ds(q * _QB + i, 16)]
                outbuf[pl.ds(i, 16)] = plsc.load_gather(planebuf, [iv])

            inflight[qb] = pltpu.async_copy(
                outbuf, out.at[pid, pl.ds(q * _QB, _QB)], sems[qb])
            qcount += 1
    for cp in inflight:
        if cp is not None:
            cp.wait()


def kernel(x, tables):
    tabt = tables.transpose(0, 2, 1)
    xt = x.astype(jnp.int32).T
    out_t = _sc_plane_gather(tabt, xt)
    return out_t.T
